# Initial kernel scaffold; baseline (speedup 1.0000x reference)
import jax
import jax.numpy as jnp
from jax.experimental import pallas as pl


def kernel(input_ids, word_embeddings):
    raise NotImplementedError("write your pallas kernel here")


# speedup vs baseline: 1.9019x; 1.9019x over previous
import functools

import jax
import jax.numpy as jnp
from jax import lax
from jax.experimental import pallas as pl
from jax.experimental.pallas import tpu as pltpu
from jax.experimental.pallas import tpu_sc as plsc

_W = 64


def _sc_gather(table, idx_flat):
    B = idx_flat.shape[0]
    D = table.shape[1]
    info = plsc.get_sparse_core_info()
    nw = info.num_cores * info.num_subcores
    b_per_w = B // nw
    nchunks = b_per_w // _W
    assert b_per_w % _W == 0 and nchunks % 2 == 0
    mesh = plsc.VectorSubcoreMesh(core_axis_name="c", subcore_axis_name="s")

    @functools.partial(
        pl.kernel,
        out_type=jax.ShapeDtypeStruct((B, D), table.dtype),
        mesh=mesh,
        scratch_types=[
            pltpu.VMEM((b_per_w,), jnp.int32),
            pltpu.VMEM((_W, D), jnp.float32),
            pltpu.VMEM((_W, D), jnp.float32),
            pltpu.SemaphoreType.DMA,
            pltpu.SemaphoreType.DMA,
            pltpu.SemaphoreType.DMA,
            pltpu.SemaphoreType.DMA,
        ],
    )
    def k(table_hbm, idx_hbm, out_hbm, idx_v, rows0, rows1, g0, g1, w0, w1):
        wid = lax.axis_index("s") * info.num_cores + lax.axis_index("c")
        base = wid * b_per_w
        pltpu.sync_copy(idx_hbm.at[pl.ds(base, b_per_w)], idx_v)

        def start_gather(c, buf, sem):
            pltpu.async_copy(table_hbm.at[idx_v.at[pl.ds(c * _W, _W)]], buf, sem)

        def wait_gather(buf, sem):
            pltpu.make_async_copy(table_hbm.at[idx_v.at[pl.ds(0, _W)]], buf, sem).wait()

        def start_write(c, buf, sem):
            pltpu.async_copy(buf, out_hbm.at[pl.ds(base + c * _W, _W)], sem)

        def wait_write(buf, sem):
            pltpu.make_async_copy(buf, out_hbm.at[pl.ds(base, _W)], sem).wait()

        start_gather(0, rows0, g0)
        start_gather(1, rows1, g1)
        wait_gather(rows0, g0)
        start_write(0, rows0, w0)
        wait_gather(rows1, g1)
        start_write(1, rows1, w1)

        @pl.loop(2, nchunks, step=2)
        def _(c):
            wait_write(rows0, w0)
            start_gather(c, rows0, g0)
            wait_write(rows1, w1)
            start_gather(c + 1, rows1, g1)
            wait_gather(rows0, g0)
            start_write(c, rows0, w0)
            wait_gather(rows1, g1)
            start_write(c + 1, rows1, w1)

        wait_write(rows0, w0)
        wait_write(rows1, w1)

    return k(table, idx_flat)


def kernel(input_ids, word_embeddings):
    s0, s1 = input_ids.shape
    idx_flat = input_ids.reshape(-1).astype(jnp.int32)
    out = _sc_gather(word_embeddings, idx_flat)
    return out.reshape(s0, s1, word_embeddings.shape[1])


# speedup vs baseline: 1.9068x; 1.0026x over previous
import functools

import jax
import jax.numpy as jnp
from jax import lax
from jax.experimental import pallas as pl
from jax.experimental.pallas import tpu as pltpu
from jax.experimental.pallas import tpu_sc as plsc

_W = 80


def _sc_gather(table, idx_flat):
    B = idx_flat.shape[0]
    D = table.shape[1]
    info = plsc.get_sparse_core_info()
    nw = info.num_cores * info.num_subcores
    b_per_w = B // nw
    nchunks = b_per_w // _W
    assert b_per_w % _W == 0 and nchunks % 2 == 0
    mesh = plsc.VectorSubcoreMesh(core_axis_name="c", subcore_axis_name="s")

    @functools.partial(
        pl.kernel,
        out_type=jax.ShapeDtypeStruct((B, D), table.dtype),
        mesh=mesh,
        scratch_types=[
            pltpu.VMEM((b_per_w,), jnp.int32),
            pltpu.VMEM((_W, D), jnp.float32),
            pltpu.VMEM((_W, D), jnp.float32),
            pltpu.SemaphoreType.DMA,
            pltpu.SemaphoreType.DMA,
            pltpu.SemaphoreType.DMA,
            pltpu.SemaphoreType.DMA,
        ],
    )
    def k(table_hbm, idx_hbm, out_hbm, idx_v, rows0, rows1, g0, g1, w0, w1):
        wid = lax.axis_index("s") * info.num_cores + lax.axis_index("c")
        base = wid * b_per_w
        pltpu.sync_copy(idx_hbm.at[pl.ds(base, b_per_w)], idx_v)

        def start_gather(c, buf, sem):
            pltpu.async_copy(table_hbm.at[idx_v.at[pl.ds(c * _W, _W)]], buf, sem)

        def wait_gather(buf, sem):
            pltpu.make_async_copy(table_hbm.at[idx_v.at[pl.ds(0, _W)]], buf, sem).wait()

        def start_write(c, buf, sem):
            pltpu.async_copy(buf, out_hbm.at[pl.ds(base + c * _W, _W)], sem)

        def wait_write(buf, sem):
            pltpu.make_async_copy(buf, out_hbm.at[pl.ds(base, _W)], sem).wait()

        start_gather(0, rows0, g0)
        start_gather(1, rows1, g1)
        wait_gather(rows0, g0)
        start_write(0, rows0, w0)
        wait_gather(rows1, g1)
        start_write(1, rows1, w1)

        @pl.loop(2, nchunks, step=2)
        def _(c):
            wait_write(rows0, w0)
            start_gather(c, rows0, g0)
            wait_write(rows1, w1)
            start_gather(c + 1, rows1, g1)
            wait_gather(rows0, g0)
            start_write(c, rows0, w0)
            wait_gather(rows1, g1)
            start_write(c + 1, rows1, w1)

        wait_write(rows0, w0)
        wait_write(rows1, w1)

    return k(table, idx_flat)


def kernel(input_ids, word_embeddings):
    s0, s1 = input_ids.shape
    idx_flat = input_ids.reshape(-1).astype(jnp.int32)
    out = _sc_gather(word_embeddings, idx_flat)
    return out.reshape(s0, s1, word_embeddings.shape[1])


# speedup vs baseline: 1.9121x; 1.0027x over previous
import functools

import jax
import jax.numpy as jnp
from jax import lax
from jax.experimental import pallas as pl
from jax.experimental.pallas import tpu as pltpu
from jax.experimental.pallas import tpu_sc as plsc

_W = 40
_NBUF = 4


def _sc_gather(table, idx_flat):
    B = idx_flat.shape[0]
    D = table.shape[1]
    info = plsc.get_sparse_core_info()
    nw = info.num_cores * info.num_subcores
    b_per_w = B // nw
    nchunks = b_per_w // _W
    assert b_per_w % _W == 0 and nchunks % _NBUF == 0
    mesh = plsc.VectorSubcoreMesh(core_axis_name="c", subcore_axis_name="s")

    @functools.partial(
        pl.kernel,
        out_type=jax.ShapeDtypeStruct((B, D), table.dtype),
        mesh=mesh,
        scratch_types=[pltpu.VMEM((b_per_w,), jnp.int32)]
        + [pltpu.VMEM((_W, D), jnp.float32)] * _NBUF
        + [pltpu.SemaphoreType.DMA] * (2 * _NBUF),
    )
    def k(table_hbm, idx_hbm, out_hbm, idx_v, *bufs_sems):
        bufs = bufs_sems[:_NBUF]
        gsems = bufs_sems[_NBUF : 2 * _NBUF]
        wsems = bufs_sems[2 * _NBUF :]
        wid = lax.axis_index("s") * info.num_cores + lax.axis_index("c")
        base = wid * b_per_w
        pltpu.sync_copy(idx_hbm.at[pl.ds(base, b_per_w)], idx_v)

        def start_gather(c, b):
            pltpu.async_copy(
                table_hbm.at[idx_v.at[pl.ds(c * _W, _W)]], bufs[b], gsems[b]
            )

        def wait_gather(b):
            pltpu.make_async_copy(
                table_hbm.at[idx_v.at[pl.ds(0, _W)]], bufs[b], gsems[b]
            ).wait()

        def start_write(c, b):
            pltpu.async_copy(bufs[b], out_hbm.at[pl.ds(base + c * _W, _W)], wsems[b])

        def wait_write(b):
            pltpu.make_async_copy(
                bufs[b], out_hbm.at[pl.ds(base, _W)], wsems[b]
            ).wait()

        for b in range(_NBUF):
            start_gather(b, b)
        for b in range(_NBUF):
            wait_gather(b)
            start_write(b, b)

        @pl.loop(_NBUF, nchunks, step=_NBUF)
        def _(c):
            for b in range(_NBUF):
                wait_write(b)
                start_gather(c + b, b)
            for b in range(_NBUF):
                wait_gather(b)
                start_write(c + b, b)

        for b in range(_NBUF):
            wait_write(b)

    return k(table, idx_flat)


def kernel(input_ids, word_embeddings):
    s0, s1 = input_ids.shape
    idx_flat = input_ids.reshape(-1).astype(jnp.int32)
    out = _sc_gather(word_embeddings, idx_flat)
    return out.reshape(s0, s1, word_embeddings.shape[1])
